# baseline (speedup 1.0000x reference)
import functools

import jax
import jax.numpy as jnp
from jax import lax
from jax.experimental import pallas as pl
from jax.experimental.pallas import tpu as pltpu
from jax.experimental.pallas import tpu_sc as plsc

_N = 10000
_E = 320000
_D = 128
_G = 64
_C = 10

_NPAD = 10240
_NW = 32
_EPT = _E // _NW
_CH = 80
_NCH = _EPT // _CH
_RPT = _NPAD // 16


def _sc_mesh():
    return plsc.VectorSubcoreMesh(core_axis_name="c", subcore_axis_name="s")


@functools.partial(
    pl.kernel,
    out_type=jax.ShapeDtypeStruct((2, _NPAD), jnp.float32),
    mesh=_sc_mesh(),
    scratch_types=[
        pltpu.VMEM((_NCH, _CH), jnp.int32),
        pltpu.VMEM((_CH,), jnp.float32),
        pltpu.VMEM((_RPT,), jnp.float32),
        pltpu.VMEM_SHARED((_NPAD,), jnp.float32),
    ],
)
def _deg_kernel(dst_hbm, out_hbm, dst_v, ones_v, tmp_v, acc_sh):
    cid = lax.axis_index("c")
    sid = lax.axis_index("s")
    wid = cid * 16 + sid

    for j in range(_CH // 16):
        ones_v[pl.ds(j * 16, 16)] = jnp.ones((16,), jnp.float32)

    def _zero(i, _):
        tmp_v[pl.ds(i * 16, 16)] = jnp.zeros((16,), jnp.float32)
        return 0
    lax.fori_loop(0, _RPT // 16, _zero, 0)
    pltpu.sync_copy(tmp_v, acc_sh.at[pl.ds(sid * _RPT, _RPT)])
    plsc.subcore_barrier()

    pltpu.sync_copy(dst_hbm.at[wid], dst_v)

    def _body(c, _):
        pltpu.sync_copy(ones_v, acc_sh.at[dst_v.at[c]], add=True)
        return 0
    lax.fori_loop(0, _NCH, _body, 0)
    plsc.subcore_barrier()

    pltpu.sync_copy(acc_sh.at[pl.ds(sid * _RPT, _RPT)], tmp_v)
    pltpu.sync_copy(tmp_v, out_hbm.at[cid, pl.ds(sid * _RPT, _RPT)])


_CSZ = 125
_CN = _EPT // _CSZ


@functools.partial(
    pl.kernel,
    out_type=jax.ShapeDtypeStruct((2, _NPAD, _D), jnp.float32),
    mesh=_sc_mesh(),
    scratch_types=[
        pltpu.VMEM((2, _CSZ), jnp.int32),
        pltpu.VMEM((2, _CSZ), jnp.int32),
        pltpu.VMEM((_CSZ, _D), jnp.float32),
        pltpu.VMEM((_CSZ, _D), jnp.float32),
        pltpu.VMEM_SHARED((_NPAD, _D), jnp.float32),
        pltpu.SemaphoreType.DMA,
        pltpu.SemaphoreType.DMA,
        pltpu.SemaphoreType.DMA,
        pltpu.SemaphoreType.DMA,
        pltpu.SemaphoreType.DMA,
        pltpu.SemaphoreType.DMA,
    ],
)
def _mp_kernel(hs_hbm, src_hbm, dst_hbm, out_hbm, srcb, dstb, rows0, rows1,
               acc_sh, gsem0, gsem1, ssem0, ssem1, dsem0, dsem1):
    cid = lax.axis_index("c")
    sid = lax.axis_index("s")
    wid = cid * 16 + sid

    def _zrow(i, _):
        for j in range(_D // 16):
            rows0[i, pl.ds(j * 16, 16)] = jnp.zeros((16,), jnp.float32)
        return 0
    lax.fori_loop(0, _CH, _zrow, 0)

    def _zacc(k, _):
        pltpu.sync_copy(rows0.at[pl.ds(0, _CH)],
                        acc_sh.at[pl.ds(sid * _RPT + k * _CH, _CH)])
        return 0
    lax.fori_loop(0, _RPT // _CH, _zacc, 0)
    plsc.subcore_barrier()

    pltpu.sync_copy(src_hbm.at[wid, 0], srcb.at[0])
    pltpu.async_copy(hs_hbm.at[srcb.at[0]], rows0, gsem0)
    pltpu.async_copy(src_hbm.at[wid, 1], srcb.at[1], ssem1)
    pltpu.async_copy(dst_hbm.at[wid, 0], dstb.at[0], dsem0)

    def _step(c, rows_c, rows_n, gsem_c, gsem_n, ssem_c, ssem_n, dsem_c,
              dsem_n, pc, pn):
        pltpu.make_async_copy(hs_hbm.at[srcb.at[pc]], rows_c, gsem_c).wait()

        @pl.when(c < _CN - 1)
        def _():
            pltpu.make_async_copy(src_hbm.at[wid, 0], srcb.at[pn],
                                  ssem_n).wait()
            pltpu.async_copy(hs_hbm.at[srcb.at[pn]], rows_n, gsem_n)
            pltpu.async_copy(dst_hbm.at[wid, c + 1], dstb.at[pn], dsem_n)

        @pl.when(c < _CN - 2)
        def _():
            pltpu.async_copy(src_hbm.at[wid, c + 2], srcb.at[pc], ssem_c)

        pltpu.make_async_copy(dst_hbm.at[wid, 0], dstb.at[pc], dsem_c).wait()
        pltpu.sync_copy(rows_c, acc_sh.at[dstb.at[pc]], add=True)

    def _body(c, _):
        @pl.when(c % 2 == 0)
        def _():
            _step(c, rows0, rows1, gsem0, gsem1, ssem0, ssem1, dsem0, dsem1,
                  0, 1)

        @pl.when(c % 2 == 1)
        def _():
            _step(c, rows1, rows0, gsem1, gsem0, ssem1, ssem0, dsem1, dsem0,
                  1, 0)
        return 0
    lax.fori_loop(0, _CN, _body, 0)
    plsc.subcore_barrier()

    def _out(k, _):
        pltpu.sync_copy(acc_sh.at[pl.ds(sid * _RPT + k * _CH, _CH)],
                        rows0.at[pl.ds(0, _CH)])
        pltpu.sync_copy(rows0.at[pl.ds(0, _CH)],
                        out_hbm.at[cid, pl.ds(sid * _RPT + k * _CH, _CH)])
        return 0
    lax.fori_loop(0, _RPT // _CH, _out, 0)


_BR = 256
_NB = _NPAD // _BR


def _tc1_body(x_ref, w_ref, dp_ref, hs_ref, dinv_ref):
    deg = dp_ref[0, :] + dp_ref[1, :] + 1.0
    dinv = lax.rsqrt(deg)
    hl = jnp.dot(x_ref[...], w_ref[...], preferred_element_type=jnp.float32)
    hs_ref[...] = hl * dinv[:, None]
    dinv_ref[...] = dinv


def _tc1(x_pad, W1, degp):
    return pl.pallas_call(
        _tc1_body,
        grid=(_NB,),
        in_specs=[
            pl.BlockSpec((_BR, _D), lambda i: (i, 0)),
            pl.BlockSpec((_D, _D), lambda i: (0, 0)),
            pl.BlockSpec((2, _BR), lambda i: (0, i)),
        ],
        out_specs=[
            pl.BlockSpec((_BR, _D), lambda i: (i, 0)),
            pl.BlockSpec((_BR,), lambda i: (i,)),
        ],
        out_shape=[
            jax.ShapeDtypeStruct((_NPAD, _D), jnp.float32),
            jax.ShapeDtypeStruct((_NPAD,), jnp.float32),
        ],
    )(x_pad, W1, degp)


def _tc2_body(p_ref, hs_ref, dinv_ref, b1_ref, g_ref, bb_ref, w2_ref, out_ref):
    dinv = dinv_ref[...]
    conv = (p_ref[0] + p_ref[1] + hs_ref[...]) * dinv[:, None] + b1_ref[...]
    h = jnp.maximum(conv, 0.0)
    mu = jnp.mean(h, axis=-1, keepdims=True)
    var = jnp.mean((h - mu) ** 2, axis=-1, keepdims=True)
    hn = (h - mu) / jnp.sqrt(var + 1e-5) * g_ref[...] + bb_ref[...]
    hl2 = jnp.dot(hn, w2_ref[...], preferred_element_type=jnp.float32)
    out_ref[...] = hl2 * dinv[:, None]


def _tc2(p, hs1, dinv, b1, ln_g, ln_b, W2):
    return pl.pallas_call(
        _tc2_body,
        grid=(_NB,),
        in_specs=[
            pl.BlockSpec((2, _BR, _D), lambda i: (0, i, 0)),
            pl.BlockSpec((_BR, _D), lambda i: (i, 0)),
            pl.BlockSpec((_BR,), lambda i: (i,)),
            pl.BlockSpec((_D,), lambda i: (0,)),
            pl.BlockSpec((_D,), lambda i: (0,)),
            pl.BlockSpec((_D,), lambda i: (0,)),
            pl.BlockSpec((_D, _D), lambda i: (0, 0)),
        ],
        out_specs=pl.BlockSpec((_BR, _D), lambda i: (i, 0)),
        out_shape=jax.ShapeDtypeStruct((_NPAD, _D), jnp.float32),
    )(p, hs1, dinv, b1, ln_g, ln_b, W2)


def _tc3_body(p_ref, hs_ref, dinv_ref, b2_ref, batch_ref, w3_ref, b3_ref,
              w4_ref, b4_ref, emb_ref, ls_ref, pool_acc):
    i = pl.program_id(0)
    dinv = dinv_ref[...]
    emb = (p_ref[0] + p_ref[1] + hs_ref[...]) * dinv[:, None] + b2_ref[...]
    emb_ref[...] = emb
    hr = jnp.maximum(emb, 0.0)
    b = batch_ref[...]
    onehot = (b[None, :] == lax.broadcasted_iota(jnp.int32, (_G, _BR), 0)
              ).astype(jnp.float32)
    contrib = jnp.dot(onehot, hr, preferred_element_type=jnp.float32)

    @pl.when(i == 0)
    def _():
        pool_acc[...] = contrib

    @pl.when(i > 0)
    def _():
        pool_acc[...] = pool_acc[...] + contrib

    @pl.when(i == _NB - 1)
    def _():
        z = jnp.dot(pool_acc[...], w3_ref[...],
                    preferred_element_type=jnp.float32) + b3_ref[...]
        z = jnp.dot(z, w4_ref[...],
                    preferred_element_type=jnp.float32) + b4_ref[...]
        m = jnp.max(z, axis=-1, keepdims=True)
        ls_ref[...] = (z - m) - jnp.log(
            jnp.sum(jnp.exp(z - m), axis=-1, keepdims=True))


def _tc3(p, hs2, dinv, b2, batch_pad, W3, b3, W4, b4):
    return pl.pallas_call(
        _tc3_body,
        grid=(_NB,),
        in_specs=[
            pl.BlockSpec((2, _BR, _D), lambda i: (0, i, 0)),
            pl.BlockSpec((_BR, _D), lambda i: (i, 0)),
            pl.BlockSpec((_BR,), lambda i: (i,)),
            pl.BlockSpec((_D,), lambda i: (0,)),
            pl.BlockSpec((_BR,), lambda i: (i,)),
            pl.BlockSpec((_D, _D), lambda i: (0, 0)),
            pl.BlockSpec((_D,), lambda i: (0,)),
            pl.BlockSpec((_D, _C), lambda i: (0, 0)),
            pl.BlockSpec((_C,), lambda i: (0,)),
        ],
        out_specs=[
            pl.BlockSpec((_BR, _D), lambda i: (i, 0)),
            pl.BlockSpec((_G, _C), lambda i: (0, 0)),
        ],
        out_shape=[
            jax.ShapeDtypeStruct((_N, _D), jnp.float32),
            jax.ShapeDtypeStruct((_G, _C), jnp.float32),
        ],
        scratch_shapes=[pltpu.VMEM((_G, _D), jnp.float32)],
    )(p, hs2, dinv, b2, batch_pad, W3, b3, W4, b4)


def kernel(x, edge_index, batch, W1, b1, W2, b2, ln_g, ln_b, W3, b3, W4, b4):
    x_pad = jnp.pad(x, ((0, _NPAD - _N), (0, 0)))
    src = edge_index[0].reshape(_NW, _CN, _CSZ)
    dst = edge_index[1].reshape(_NW, _CN, _CSZ)
    deg_dst = edge_index[1].reshape(_NW, _NCH, _CH)
    batch_pad = jnp.pad(batch, (0, _NPAD - _N), constant_values=_G)

    degp = _deg_kernel(deg_dst)
    hs1, dinv = _tc1(x_pad, W1, degp)
    p1 = _mp_kernel(hs1, src, dst)
    hs2 = _tc2(p1, hs1, dinv, b1, ln_g, ln_b, W2)
    p2 = _mp_kernel(hs2, src, dst)
    emb, ls = _tc3(p2, hs2, dinv, b2, batch_pad, W3, b3, W4, b4)
    return emb, ls


# speedup vs baseline: 1.0386x; 1.0386x over previous
import functools

import jax
import jax.numpy as jnp
from jax import lax
from jax.experimental import pallas as pl
from jax.experimental.pallas import tpu as pltpu
from jax.experimental.pallas import tpu_sc as plsc

_N = 10000
_E = 320000
_D = 128
_G = 64
_C = 10

_NPAD = 10240
_NW = 32
_EPT = _E // _NW
_CH = 80
_NCH = _EPT // _CH
_RPT = _NPAD // 16


def _sc_mesh():
    return plsc.VectorSubcoreMesh(core_axis_name="c", subcore_axis_name="s")


@functools.partial(
    pl.kernel,
    out_type=jax.ShapeDtypeStruct((2, _NPAD), jnp.float32),
    mesh=_sc_mesh(),
    scratch_types=[
        pltpu.VMEM((_NCH, _CH), jnp.int32),
        pltpu.VMEM((_CH,), jnp.float32),
        pltpu.VMEM((_RPT,), jnp.float32),
        pltpu.VMEM_SHARED((_NPAD,), jnp.float32),
    ],
)
def _deg_kernel(dst_hbm, out_hbm, dst_v, ones_v, tmp_v, acc_sh):
    cid = lax.axis_index("c")
    sid = lax.axis_index("s")
    wid = cid * 16 + sid

    for j in range(_CH // 16):
        ones_v[pl.ds(j * 16, 16)] = jnp.ones((16,), jnp.float32)

    def _zero(i, _):
        tmp_v[pl.ds(i * 16, 16)] = jnp.zeros((16,), jnp.float32)
        return 0
    lax.fori_loop(0, _RPT // 16, _zero, 0)
    pltpu.sync_copy(tmp_v, acc_sh.at[pl.ds(sid * _RPT, _RPT)])
    plsc.subcore_barrier()

    pltpu.sync_copy(dst_hbm.at[wid], dst_v)

    def _body(c, _):
        pltpu.sync_copy(ones_v, acc_sh.at[dst_v.at[c]], add=True)
        return 0
    lax.fori_loop(0, _NCH, _body, 0)
    plsc.subcore_barrier()

    pltpu.sync_copy(acc_sh.at[pl.ds(sid * _RPT, _RPT)], tmp_v)
    pltpu.sync_copy(tmp_v, out_hbm.at[cid, pl.ds(sid * _RPT, _RPT)])


_CSZ = 100
_CN = _EPT // _CSZ


@functools.partial(
    pl.kernel,
    out_type=jax.ShapeDtypeStruct((2, _NPAD, _D), jnp.float32),
    mesh=_sc_mesh(),
    scratch_types=[
        pltpu.VMEM((_CN, _CSZ), jnp.int32),
        pltpu.VMEM((2, _CSZ), jnp.int32),
        pltpu.VMEM((_CSZ, _D), jnp.float32),
        pltpu.VMEM((_CSZ, _D), jnp.float32),
        pltpu.VMEM_SHARED((_NPAD, _D), jnp.float32),
        pltpu.SemaphoreType.DMA,
        pltpu.SemaphoreType.DMA,
        pltpu.SemaphoreType.DMA,
        pltpu.SemaphoreType.DMA,
    ],
)
def _mp_kernel(hs_hbm, src_hbm, dst_hbm, out_hbm, src_v, dstb, rows0, rows1,
               acc_sh, sem0, sem1, semd0, semd1):
    cid = lax.axis_index("c")
    sid = lax.axis_index("s")
    wid = cid * 16 + sid

    def _zrow(i, _):
        for j in range(_D // 16):
            rows0[i, pl.ds(j * 16, 16)] = jnp.zeros((16,), jnp.float32)
        return 0
    lax.fori_loop(0, _CH, _zrow, 0)

    def _zacc(k, _):
        pltpu.sync_copy(rows0.at[pl.ds(0, _CH)],
                        acc_sh.at[pl.ds(sid * _RPT + k * _CH, _CH)])
        return 0
    lax.fori_loop(0, _RPT // _CH, _zacc, 0)
    plsc.subcore_barrier()

    pltpu.sync_copy(src_hbm.at[wid], src_v)

    pltpu.async_copy(dst_hbm.at[wid, 0], dstb.at[0], semd0)
    pltpu.async_copy(hs_hbm.at[src_v.at[0]], rows0, sem0)

    def _body(c, _):
        @pl.when(c % 2 == 0)
        def _():
            @pl.when(c < _CN - 1)
            def _():
                pltpu.async_copy(dst_hbm.at[wid, c + 1], dstb.at[1], semd1)
                pltpu.async_copy(hs_hbm.at[src_v.at[c + 1]], rows1, sem1)
            pltpu.make_async_copy(hs_hbm.at[src_v.at[c]], rows0, sem0).wait()
            pltpu.make_async_copy(dst_hbm.at[wid, c], dstb.at[0], semd0).wait()
            pltpu.sync_copy(rows0, acc_sh.at[dstb.at[0]], add=True)

        @pl.when(c % 2 == 1)
        def _():
            @pl.when(c < _CN - 1)
            def _():
                pltpu.async_copy(dst_hbm.at[wid, c + 1], dstb.at[0], semd0)
                pltpu.async_copy(hs_hbm.at[src_v.at[c + 1]], rows0, sem0)
            pltpu.make_async_copy(hs_hbm.at[src_v.at[c]], rows1, sem1).wait()
            pltpu.make_async_copy(dst_hbm.at[wid, c], dstb.at[1], semd1).wait()
            pltpu.sync_copy(rows1, acc_sh.at[dstb.at[1]], add=True)
        return 0
    lax.fori_loop(0, _CN, _body, 0)
    plsc.subcore_barrier()

    def _out(k, _):
        pltpu.sync_copy(acc_sh.at[pl.ds(sid * _RPT + k * _CH, _CH)],
                        rows0.at[pl.ds(0, _CH)])
        pltpu.sync_copy(rows0.at[pl.ds(0, _CH)],
                        out_hbm.at[cid, pl.ds(sid * _RPT + k * _CH, _CH)])
        return 0
    lax.fori_loop(0, _RPT // _CH, _out, 0)


_BR = 256
_NB = _NPAD // _BR


def _tc0_body(x_ref, w_ref, hl_ref):
    hl_ref[...] = jnp.dot(x_ref[...], w_ref[...],
                          preferred_element_type=jnp.float32)


def _tc0(x_pad, W1):
    return pl.pallas_call(
        _tc0_body,
        grid=(_NB,),
        in_specs=[
            pl.BlockSpec((_BR, _D), lambda i: (i, 0)),
            pl.BlockSpec((_D, _D), lambda i: (0, 0)),
        ],
        out_specs=pl.BlockSpec((_BR, _D), lambda i: (i, 0)),
        out_shape=jax.ShapeDtypeStruct((_NPAD, _D), jnp.float32),
    )(x_pad, W1)


def _tc1_body(hl_ref, dp_ref, hs_ref, dinv_ref):
    deg = dp_ref[0, :] + dp_ref[1, :] + 1.0
    dinv = lax.rsqrt(deg)
    hs_ref[...] = hl_ref[...] * dinv[:, None]
    dinv_ref[...] = dinv


def _tc1(hl1, degp):
    return pl.pallas_call(
        _tc1_body,
        grid=(_NB,),
        in_specs=[
            pl.BlockSpec((_BR, _D), lambda i: (i, 0)),
            pl.BlockSpec((2, _BR), lambda i: (0, i)),
        ],
        out_specs=[
            pl.BlockSpec((_BR, _D), lambda i: (i, 0)),
            pl.BlockSpec((_BR,), lambda i: (i,)),
        ],
        out_shape=[
            jax.ShapeDtypeStruct((_NPAD, _D), jnp.float32),
            jax.ShapeDtypeStruct((_NPAD,), jnp.float32),
        ],
    )(hl1, degp)


def _tc2_body(p_ref, hs_ref, dinv_ref, b1_ref, g_ref, bb_ref, w2_ref, out_ref):
    dinv = dinv_ref[...]
    conv = (p_ref[0] + p_ref[1] + hs_ref[...]) * dinv[:, None] + b1_ref[...]
    h = jnp.maximum(conv, 0.0)
    mu = jnp.mean(h, axis=-1, keepdims=True)
    var = jnp.mean((h - mu) ** 2, axis=-1, keepdims=True)
    hn = (h - mu) / jnp.sqrt(var + 1e-5) * g_ref[...] + bb_ref[...]
    hl2 = jnp.dot(hn, w2_ref[...], preferred_element_type=jnp.float32)
    out_ref[...] = hl2 * dinv[:, None]


def _tc2(p, hs1, dinv, b1, ln_g, ln_b, W2):
    return pl.pallas_call(
        _tc2_body,
        grid=(_NB,),
        in_specs=[
            pl.BlockSpec((2, _BR, _D), lambda i: (0, i, 0)),
            pl.BlockSpec((_BR, _D), lambda i: (i, 0)),
            pl.BlockSpec((_BR,), lambda i: (i,)),
            pl.BlockSpec((_D,), lambda i: (0,)),
            pl.BlockSpec((_D,), lambda i: (0,)),
            pl.BlockSpec((_D,), lambda i: (0,)),
            pl.BlockSpec((_D, _D), lambda i: (0, 0)),
        ],
        out_specs=pl.BlockSpec((_BR, _D), lambda i: (i, 0)),
        out_shape=jax.ShapeDtypeStruct((_NPAD, _D), jnp.float32),
    )(p, hs1, dinv, b1, ln_g, ln_b, W2)


def _tc3_body(p_ref, hs_ref, dinv_ref, b2_ref, batch_ref, w3_ref, b3_ref,
              w4_ref, b4_ref, emb_ref, ls_ref, pool_acc):
    i = pl.program_id(0)
    dinv = dinv_ref[...]
    emb = (p_ref[0] + p_ref[1] + hs_ref[...]) * dinv[:, None] + b2_ref[...]
    emb_ref[...] = emb
    hr = jnp.maximum(emb, 0.0)
    b = batch_ref[...]
    onehot = (b[None, :] == lax.broadcasted_iota(jnp.int32, (_G, _BR), 0)
              ).astype(jnp.float32)
    contrib = jnp.dot(onehot, hr, preferred_element_type=jnp.float32)

    @pl.when(i == 0)
    def _():
        pool_acc[...] = contrib

    @pl.when(i > 0)
    def _():
        pool_acc[...] = pool_acc[...] + contrib

    @pl.when(i == _NB - 1)
    def _():
        z = jnp.dot(pool_acc[...], w3_ref[...],
                    preferred_element_type=jnp.float32) + b3_ref[...]
        z = jnp.dot(z, w4_ref[...],
                    preferred_element_type=jnp.float32) + b4_ref[...]
        m = jnp.max(z, axis=-1, keepdims=True)
        ls_ref[...] = (z - m) - jnp.log(
            jnp.sum(jnp.exp(z - m), axis=-1, keepdims=True))


def _tc3(p, hs2, dinv, b2, batch_pad, W3, b3, W4, b4):
    return pl.pallas_call(
        _tc3_body,
        grid=(_NB,),
        in_specs=[
            pl.BlockSpec((2, _BR, _D), lambda i: (0, i, 0)),
            pl.BlockSpec((_BR, _D), lambda i: (i, 0)),
            pl.BlockSpec((_BR,), lambda i: (i,)),
            pl.BlockSpec((_D,), lambda i: (0,)),
            pl.BlockSpec((_BR,), lambda i: (i,)),
            pl.BlockSpec((_D, _D), lambda i: (0, 0)),
            pl.BlockSpec((_D,), lambda i: (0,)),
            pl.BlockSpec((_D, _C), lambda i: (0, 0)),
            pl.BlockSpec((_C,), lambda i: (0,)),
        ],
        out_specs=[
            pl.BlockSpec((_BR, _D), lambda i: (i, 0)),
            pl.BlockSpec((_G, _C), lambda i: (0, 0)),
        ],
        out_shape=[
            jax.ShapeDtypeStruct((_N, _D), jnp.float32),
            jax.ShapeDtypeStruct((_G, _C), jnp.float32),
        ],
        scratch_shapes=[pltpu.VMEM((_G, _D), jnp.float32)],
    )(p, hs2, dinv, b2, batch_pad, W3, b3, W4, b4)


def kernel(x, edge_index, batch, W1, b1, W2, b2, ln_g, ln_b, W3, b3, W4, b4):
    x_pad = jnp.pad(x, ((0, _NPAD - _N), (0, 0)))
    src = edge_index[0].reshape(_NW, _CN, _CSZ)
    dst = edge_index[1].reshape(_NW, _CN, _CSZ)
    deg_dst = edge_index[1].reshape(_NW, _NCH, _CH)
    batch_pad = jnp.pad(batch, (0, _NPAD - _N), constant_values=_G)

    degp = _deg_kernel(deg_dst)
    hl1 = _tc0(x_pad, W1)
    hs1, dinv = _tc1(hl1, degp)
    p1 = _mp_kernel(hs1, src, dst)
    hs2 = _tc2(p1, hs1, dinv, b1, ln_g, ln_b, W2)
    p2 = _mp_kernel(hs2, src, dst)
    emb, ls = _tc3(p2, hs2, dinv, b2, batch_pad, W3, b3, W4, b4)
    return emb, ls


# speedup vs baseline: 1.0999x; 1.0591x over previous
import functools

import jax
import jax.numpy as jnp
from jax import lax
from jax.experimental import pallas as pl
from jax.experimental.pallas import tpu as pltpu
from jax.experimental.pallas import tpu_sc as plsc

_N = 10000
_E = 320000
_D = 128
_G = 64
_C = 10

_NPAD = 10240
_NW = 32
_EPT = _E // _NW
_CH = 80
_NCH = _EPT // _CH
_RPT = _NPAD // 16


def _sc_mesh():
    return plsc.VectorSubcoreMesh(core_axis_name="c", subcore_axis_name="s")


@functools.partial(
    pl.kernel,
    out_type=jax.ShapeDtypeStruct((2, _NPAD), jnp.float32),
    mesh=_sc_mesh(),
    scratch_types=[
        pltpu.VMEM((_NCH, _CH), jnp.int32),
        pltpu.VMEM((_CH,), jnp.float32),
        pltpu.VMEM((_RPT,), jnp.float32),
        pltpu.VMEM_SHARED((_NPAD,), jnp.float32),
    ],
)
def _deg_kernel(dst_hbm, out_hbm, dst_v, ones_v, tmp_v, acc_sh):
    cid = lax.axis_index("c")
    sid = lax.axis_index("s")
    wid = cid * 16 + sid

    for j in range(_CH // 16):
        ones_v[pl.ds(j * 16, 16)] = jnp.ones((16,), jnp.float32)

    def _zero(i, _):
        tmp_v[pl.ds(i * 16, 16)] = jnp.zeros((16,), jnp.float32)
        return 0
    lax.fori_loop(0, _RPT // 16, _zero, 0)
    pltpu.sync_copy(tmp_v, acc_sh.at[pl.ds(sid * _RPT, _RPT)])
    plsc.subcore_barrier()

    pltpu.sync_copy(dst_hbm.at[wid], dst_v)

    def _body(c, _):
        pltpu.sync_copy(ones_v, acc_sh.at[dst_v.at[c]], add=True)
        return 0
    lax.fori_loop(0, _NCH, _body, 0)
    plsc.subcore_barrier()

    pltpu.sync_copy(acc_sh.at[pl.ds(sid * _RPT, _RPT)], tmp_v)
    pltpu.sync_copy(tmp_v, out_hbm.at[cid, pl.ds(sid * _RPT, _RPT)])


_CSZ = 100
_CN = _EPT // _CSZ


@functools.partial(
    pl.kernel,
    out_type=jax.ShapeDtypeStruct((2, _NPAD, _D), jnp.float32),
    mesh=_sc_mesh(),
    scratch_types=[
        pltpu.VMEM((_CN, _CSZ), jnp.int32),
        pltpu.VMEM((2, _CSZ), jnp.int32),
        pltpu.VMEM((_CSZ, _D), jnp.float32),
        pltpu.VMEM((_CSZ, _D), jnp.float32),
        pltpu.VMEM_SHARED((_NPAD, _D), jnp.float32),
        pltpu.SemaphoreType.DMA,
        pltpu.SemaphoreType.DMA,
        pltpu.SemaphoreType.DMA,
        pltpu.SemaphoreType.DMA,
    ],
)
def _mp_kernel(hs_hbm, src_hbm, dst_hbm, out_hbm, src_v, dstb, rows0, rows1,
               acc_sh, sem0, sem1, semd0, semd1):
    cid = lax.axis_index("c")
    sid = lax.axis_index("s")
    wid = cid * 16 + sid

    def _zrow(i, _):
        for j in range(_D // 16):
            rows0[i, pl.ds(j * 16, 16)] = jnp.zeros((16,), jnp.float32)
        return 0
    lax.fori_loop(0, _CH, _zrow, 0)

    def _zacc(k, _):
        pltpu.sync_copy(rows0.at[pl.ds(0, _CH)],
                        acc_sh.at[pl.ds(sid * _RPT + k * _CH, _CH)])
        return 0
    lax.fori_loop(0, _RPT // _CH, _zacc, 0)
    plsc.subcore_barrier()

    pltpu.sync_copy(src_hbm.at[wid], src_v)

    pltpu.async_copy(dst_hbm.at[wid, 0], dstb.at[0], semd0)
    pltpu.async_copy(hs_hbm.at[src_v.at[0]], rows0, sem0)

    def _body(c, _):
        @pl.when(c % 2 == 0)
        def _():
            @pl.when(c < _CN - 1)
            def _():
                pltpu.async_copy(dst_hbm.at[wid, c + 1], dstb.at[1], semd1)
                pltpu.async_copy(hs_hbm.at[src_v.at[c + 1]], rows1, sem1)
            pltpu.make_async_copy(hs_hbm.at[src_v.at[c]], rows0, sem0).wait()
            pltpu.make_async_copy(dst_hbm.at[wid, c], dstb.at[0], semd0).wait()
            pltpu.sync_copy(rows0, acc_sh.at[dstb.at[0]], add=True)

        @pl.when(c % 2 == 1)
        def _():
            @pl.when(c < _CN - 1)
            def _():
                pltpu.async_copy(dst_hbm.at[wid, c + 1], dstb.at[0], semd0)
                pltpu.async_copy(hs_hbm.at[src_v.at[c + 1]], rows0, sem0)
            pltpu.make_async_copy(hs_hbm.at[src_v.at[c]], rows1, sem1).wait()
            pltpu.make_async_copy(dst_hbm.at[wid, c], dstb.at[1], semd1).wait()
            pltpu.sync_copy(rows1, acc_sh.at[dstb.at[1]], add=True)
        return 0
    lax.fori_loop(0, _CN, _body, 0)
    plsc.subcore_barrier()

    def _out(k, _):
        pltpu.sync_copy(acc_sh.at[pl.ds(sid * _RPT + k * _CH, _CH)],
                        out_hbm.at[cid, pl.ds(sid * _RPT + k * _CH, _CH)])
        return 0
    lax.fori_loop(0, _RPT // _CH, _out, 0)


_BR = 256
_NB = _NPAD // _BR


def _tc1_body(x_ref, w_ref, dp_ref, hs_ref, dinv_ref):
    deg = dp_ref[0, :] + dp_ref[1, :] + 1.0
    dinv = lax.rsqrt(deg)
    hl = jnp.dot(x_ref[...], w_ref[...], preferred_element_type=jnp.float32)
    hs_ref[...] = hl * dinv[:, None]
    dinv_ref[...] = dinv


def _tc1(x_pad, W1, degp):
    return pl.pallas_call(
        _tc1_body,
        grid=(_NB,),
        in_specs=[
            pl.BlockSpec((_BR, _D), lambda i: (i, 0)),
            pl.BlockSpec((_D, _D), lambda i: (0, 0)),
            pl.BlockSpec((2, _BR), lambda i: (0, i)),
        ],
        out_specs=[
            pl.BlockSpec((_BR, _D), lambda i: (i, 0)),
            pl.BlockSpec((_BR,), lambda i: (i,)),
        ],
        out_shape=[
            jax.ShapeDtypeStruct((_NPAD, _D), jnp.float32),
            jax.ShapeDtypeStruct((_NPAD,), jnp.float32),
        ],
    )(x_pad, W1, degp)


def _tc2_body(p_ref, hs_ref, dinv_ref, b1_ref, g_ref, bb_ref, w2_ref, out_ref):
    dinv = dinv_ref[...]
    conv = (p_ref[0] + p_ref[1] + hs_ref[...]) * dinv[:, None] + b1_ref[...]
    h = jnp.maximum(conv, 0.0)
    mu = jnp.mean(h, axis=-1, keepdims=True)
    var = jnp.mean((h - mu) ** 2, axis=-1, keepdims=True)
    hn = (h - mu) / jnp.sqrt(var + 1e-5) * g_ref[...] + bb_ref[...]
    hl2 = jnp.dot(hn, w2_ref[...], preferred_element_type=jnp.float32)
    out_ref[...] = hl2 * dinv[:, None]


def _tc2(p, hs1, dinv, b1, ln_g, ln_b, W2):
    return pl.pallas_call(
        _tc2_body,
        grid=(_NB,),
        in_specs=[
            pl.BlockSpec((2, _BR, _D), lambda i: (0, i, 0)),
            pl.BlockSpec((_BR, _D), lambda i: (i, 0)),
            pl.BlockSpec((_BR,), lambda i: (i,)),
            pl.BlockSpec((_D,), lambda i: (0,)),
            pl.BlockSpec((_D,), lambda i: (0,)),
            pl.BlockSpec((_D,), lambda i: (0,)),
            pl.BlockSpec((_D, _D), lambda i: (0, 0)),
        ],
        out_specs=pl.BlockSpec((_BR, _D), lambda i: (i, 0)),
        out_shape=jax.ShapeDtypeStruct((_NPAD, _D), jnp.float32),
    )(p, hs1, dinv, b1, ln_g, ln_b, W2)


def _tc3_body(p_ref, hs_ref, dinv_ref, b2_ref, batch_ref, w3_ref, b3_ref,
              w4_ref, b4_ref, emb_ref, ls_ref, pool_acc):
    i = pl.program_id(0)
    dinv = dinv_ref[...]
    emb = (p_ref[0] + p_ref[1] + hs_ref[...]) * dinv[:, None] + b2_ref[...]
    emb_ref[...] = emb
    hr = jnp.maximum(emb, 0.0)
    b = batch_ref[...]
    onehot = (b[None, :] == lax.broadcasted_iota(jnp.int32, (_G, _BR), 0)
              ).astype(jnp.float32)
    contrib = jnp.dot(onehot, hr, preferred_element_type=jnp.float32)

    @pl.when(i == 0)
    def _():
        pool_acc[...] = contrib

    @pl.when(i > 0)
    def _():
        pool_acc[...] = pool_acc[...] + contrib

    @pl.when(i == _NB - 1)
    def _():
        z = jnp.dot(pool_acc[...], w3_ref[...],
                    preferred_element_type=jnp.float32) + b3_ref[...]
        z = jnp.dot(z, w4_ref[...],
                    preferred_element_type=jnp.float32) + b4_ref[...]
        m = jnp.max(z, axis=-1, keepdims=True)
        ls_ref[...] = (z - m) - jnp.log(
            jnp.sum(jnp.exp(z - m), axis=-1, keepdims=True))


def _tc3(p, hs2, dinv, b2, batch_pad, W3, b3, W4, b4):
    return pl.pallas_call(
        _tc3_body,
        grid=(_NB,),
        in_specs=[
            pl.BlockSpec((2, _BR, _D), lambda i: (0, i, 0)),
            pl.BlockSpec((_BR, _D), lambda i: (i, 0)),
            pl.BlockSpec((_BR,), lambda i: (i,)),
            pl.BlockSpec((_D,), lambda i: (0,)),
            pl.BlockSpec((_BR,), lambda i: (i,)),
            pl.BlockSpec((_D, _D), lambda i: (0, 0)),
            pl.BlockSpec((_D,), lambda i: (0,)),
            pl.BlockSpec((_D, _C), lambda i: (0, 0)),
            pl.BlockSpec((_C,), lambda i: (0,)),
        ],
        out_specs=[
            pl.BlockSpec((_BR, _D), lambda i: (i, 0)),
            pl.BlockSpec((_G, _C), lambda i: (0, 0)),
        ],
        out_shape=[
            jax.ShapeDtypeStruct((_N, _D), jnp.float32),
            jax.ShapeDtypeStruct((_G, _C), jnp.float32),
        ],
        scratch_shapes=[pltpu.VMEM((_G, _D), jnp.float32)],
    )(p, hs2, dinv, b2, batch_pad, W3, b3, W4, b4)


def kernel(x, edge_index, batch, W1, b1, W2, b2, ln_g, ln_b, W3, b3, W4, b4):
    x_pad = jnp.pad(x, ((0, _NPAD - _N), (0, 0)))
    src = edge_index[0].reshape(_NW, _CN, _CSZ)
    dst = edge_index[1].reshape(_NW, _CN, _CSZ)
    deg_dst = edge_index[1].reshape(_NW, _NCH, _CH)
    batch_pad = jnp.pad(batch, (0, _NPAD - _N), constant_values=_G)

    degp = _deg_kernel(deg_dst)
    hs1, dinv = _tc1(x_pad, W1, degp)
    p1 = _mp_kernel(hs1, src, dst)
    hs2 = _tc2(p1, hs1, dinv, b1, ln_g, ln_b, W2)
    p2 = _mp_kernel(hs2, src, dst)
    emb, ls = _tc3(p2, hs2, dinv, b2, batch_pad, W3, b3, W4, b4)
    return emb, ls
